# Initial kernel scaffold; baseline (speedup 1.0000x reference)
import jax
import jax.numpy as jnp
from jax.experimental import pallas as pl


def kernel(x, edge_index, edge_attr, W1, a_src1, a_dst1, a_e1, We1, b1, W2, a_src2, a_dst2, a_e2, We2, b2, W3, a_src3, a_dst3, a_e3, We3, b3, Wfc, bfc):
    raise NotImplementedError("write your pallas kernel here")


# speedup vs baseline: 1.1099x; 1.1099x over previous
import functools

import jax
import jax.numpy as jnp
from jax.experimental import pallas as pl
from jax.experimental.pallas import tpu as pltpu


def _fc_body(h_ref, w_ref, b_ref, o_ref):
    o_ref[...] = jnp.dot(h_ref[...], w_ref[...],
                         preferred_element_type=jnp.float32) + b_ref[...]


def _fc(h, Wfc, bfc):
    n, c = h.shape
    out_dim = Wfc.shape[1]
    BN = 2000
    return pl.pallas_call(
        _fc_body,
        grid=(n // BN,),
        in_specs=[
            pl.BlockSpec((BN, c), lambda i: (i, 0)),
            pl.BlockSpec((c, out_dim), lambda i: (0, 0)),
            pl.BlockSpec((1, out_dim), lambda i: (0, 0)),
        ],
        out_specs=pl.BlockSpec((BN, out_dim), lambda i: (i, 0)),
        out_shape=jax.ShapeDtypeStruct((n, out_dim), jnp.float32),
    )(h, Wfc, bfc.reshape(1, out_dim))


def _gat_conv(x, src, dst, ea, W, a_src, a_dst, a_edge, We, b, heads, ch, concat):
    n = x.shape[0]
    h = (x @ W).reshape(n, heads, ch)
    alpha_src = (h * a_src).sum(-1)
    alpha_dst = (h * a_dst).sum(-1)
    he = (ea @ We).reshape(-1, heads, ch)
    alpha_edge = (he * a_edge).sum(-1)
    alpha = alpha_src[src] + alpha_dst[dst] + alpha_edge
    alpha = jax.nn.leaky_relu(alpha, 0.2)
    ex = jnp.exp(alpha)
    den = jax.ops.segment_sum(ex, dst, num_segments=n)
    msg = h[src] * ex[:, :, None]
    out = jax.ops.segment_sum(msg, dst, num_segments=n)
    out = out / (den[:, :, None] + 1e-16)
    if concat:
        out = out.reshape(n, heads * ch)
    else:
        out = out.mean(axis=1)
    return out + b


def kernel(x, edge_index, edge_attr,
           W1, a_src1, a_dst1, a_e1, We1, b1,
           W2, a_src2, a_dst2, a_e2, We2, b2,
           W3, a_src3, a_dst3, a_e3, We3, b3,
           Wfc, bfc):
    H, C = 4, 16
    src = edge_index[0]
    dst = edge_index[1]
    h = _gat_conv(x, src, dst, edge_attr, W1, a_src1, a_dst1, a_e1, We1, b1, H, C, True)
    h = jax.nn.relu(h)
    h = _gat_conv(h, src, dst, edge_attr, W2, a_src2, a_dst2, a_e2, We2, b2, H, C, True)
    h = jax.nn.relu(h)
    h = _gat_conv(h, src, dst, edge_attr, W3, a_src3, a_dst3, a_e3, We3, b3, 1, C, False)
    h = jax.nn.relu(h)
    return _fc(h, Wfc, bfc)


# speedup vs baseline: 45.1770x; 40.7048x over previous
import functools

import jax
import jax.numpy as jnp
from jax import lax
from jax.experimental import pallas as pl
from jax.experimental.pallas import tpu as pltpu
from jax.experimental.pallas import tpu_sc as plsc

_NC = 2
_NS = 16
_NW = _NC * _NS
_L = 16

_T = 1000


def _mesh():
    return plsc.VectorSubcoreMesh(core_axis_name="c", subcore_axis_name="s")


_SC_PARAMS = pltpu.CompilerParams(use_tc_tiling_on_sc=False)


def _split_heads(r, nh):
    outs = [r[:, 16 * h:16 * h + 16] for h in range(nh)]
    outs.append(r[:, 16 * nh:16 * nh + 16])
    outs.append(r[:, 16 * nh + 16:16 * nh + 32])
    return outs


def _node1_body(x_ref, w_ref, *out_refs):
    x = x_ref[...]
    r = x[:, 0:1] * w_ref[0:1, :] + x[:, 1:2] * w_ref[1:2, :]
    for ref, val in zip(out_refs, _split_heads(r, 4)):
        ref[...] = val


def _node1(x, wcat):
    n = x.shape[0]
    BN = 2000
    return pl.pallas_call(
        _node1_body,
        grid=(n // BN,),
        in_specs=[
            pl.BlockSpec((BN, 2), lambda i: (i, 0)),
            pl.BlockSpec((2, 96), lambda i: (0, 0)),
        ],
        out_specs=[pl.BlockSpec((BN, 16), lambda i: (i, 0))] * 6,
        out_shape=[jax.ShapeDtypeStruct((n, 16), jnp.float32)] * 6,
    )(x, wcat)


def _assemble(ps, den, b):
    d = den[0] + den[1] + 1e-16
    cols = []
    for h, p in enumerate(ps):
        cols.append((p[0] + p[1]) / d[:, h:h + 1])
    xx = jnp.concatenate(cols, axis=1)
    return jnp.maximum(xx + b, 0.0)


def _node23_body(p0, p1, p2, p3, den_ref, b_ref, w_ref, *out_refs):
    xx = _assemble([p0[...], p1[...], p2[...], p3[...]],
                   den_ref[...], b_ref[...])
    r = jnp.dot(xx, w_ref[...], preferred_element_type=jnp.float32)
    nh = (w_ref.shape[1] - 32) // 16
    for ref, val in zip(out_refs, _split_heads(r, nh)):
        ref[...] = val


def _node23(ps, den, b, wcat, nh):
    n = ps[0].shape[1]
    BN = 2000
    wcols = 16 * nh + 32
    return pl.pallas_call(
        _node23_body,
        grid=(n // BN,),
        in_specs=[pl.BlockSpec((2, BN, 16), lambda i: (0, i, 0))] * 4 + [
            pl.BlockSpec((2, BN, 16), lambda i: (0, i, 0)),
            pl.BlockSpec((1, 64), lambda i: (0, 0)),
            pl.BlockSpec((64, wcols), lambda i: (0, 0)),
        ],
        out_specs=[pl.BlockSpec((BN, 16), lambda i: (i, 0))] * (nh + 2),
        out_shape=[jax.ShapeDtypeStruct((n, 16), jnp.float32)] * (nh + 2),
    )(*ps, den, b.reshape(1, 64), wcat)


def _edge_body(ea_ref, u_ref, o1_ref, o2_ref, o3_ref):
    ea = ea_ref[...]
    r = ea[:, 0:1] * u_ref[0:1, :] + ea[:, 1:2] * u_ref[1:2, :]
    o1_ref[...] = r[:, 0:16]
    o2_ref[...] = r[:, 16:32]
    o3_ref[...] = r[:, 32:48]


def _edge_ae(ea, ucat):
    e = ea.shape[0]
    BE = 8000
    outs = [jax.ShapeDtypeStruct((e, 16), jnp.float32)] * 3
    return pl.pallas_call(
        _edge_body,
        grid=(e // BE,),
        in_specs=[
            pl.BlockSpec((BE, 2), lambda i: (i, 0)),
            pl.BlockSpec((2, 48), lambda i: (0, 0)),
        ],
        out_specs=[pl.BlockSpec((BE, 16), lambda i: (i, 0))] * 3,
        out_shape=outs,
    )(ea, ucat)


def _fc_body(p3_ref, den_ref, b3_ref, w_ref, bfc_ref, o_ref):
    acc = p3_ref[0] + p3_ref[1]
    d = den_ref[0, :, 0:1] + den_ref[1, :, 0:1] + 1e-16
    xx = jnp.maximum(acc / d + b3_ref[...], 0.0)
    o_ref[...] = jnp.dot(xx, w_ref[...],
                         preferred_element_type=jnp.float32) + bfc_ref[...]


def _fc(p3, den3, b3, wfc, bfc):
    n = p3.shape[1]
    BN = 2000
    out_dim = wfc.shape[1]
    return pl.pallas_call(
        _fc_body,
        grid=(n // BN,),
        in_specs=[
            pl.BlockSpec((2, BN, 16), lambda i: (0, i, 0)),
            pl.BlockSpec((2, BN, 16), lambda i: (0, i, 0)),
            pl.BlockSpec((1, 16), lambda i: (0, 0)),
            pl.BlockSpec((16, out_dim), lambda i: (0, 0)),
            pl.BlockSpec((1, out_dim), lambda i: (0, 0)),
        ],
        out_specs=pl.BlockSpec((BN, out_dim), lambda i: (i, 0)),
        out_shape=jax.ShapeDtypeStruct((n, out_dim), jnp.float32),
    )(p3, den3, b3.reshape(1, 16), wfc, bfc.reshape(1, out_dim))


def _write_back(sh_ref, hbm_ref, c, s, n):
    rows = (n // _NS + 7) // 8 * 8
    last = n - (_NS - 1) * rows

    @pl.when(s < _NS - 1)
    def _():
        pltpu.sync_copy(sh_ref.at[pl.ds(s * rows, rows)],
                        hbm_ref.at[c, pl.ds(s * rows, rows)])

    @pl.when(s == _NS - 1)
    def _():
        pltpu.sync_copy(sh_ref.at[pl.ds((_NS - 1) * rows, last)],
                        hbm_ref.at[c, pl.ds((_NS - 1) * rows, last)])


def _k1_call(src, dst, asrc, adst, ae, zeros16):
    n = asrc.shape[0]
    e = src.shape[0]
    ew = e // _NW
    nt = ew // _T

    def body(src_hbm, dst_hbm, asrc_hbm, adst_hbm, ae_hbm, z_hbm,
             ex_hbm, den_hbm,
             srcv, dstv, g1v, g2v, aev, exv, den_sh):
        c = lax.axis_index("c")
        s = lax.axis_index("s")

        @pl.when(s == 0)
        def _():
            pltpu.sync_copy(z_hbm, den_sh)
        plsc.subcore_barrier()

        base = (c * _NS + s) * ew

        @pl.loop(0, nt)
        def _(t):
            b = base + t * _T
            pltpu.sync_copy(src_hbm.at[pl.ds(b, _T)], srcv)
            pltpu.sync_copy(dst_hbm.at[pl.ds(b, _T)], dstv)
            pltpu.sync_copy(ae_hbm.at[pl.ds(b, _T)], aev)
            pltpu.sync_copy(asrc_hbm.at[srcv], g1v)
            pltpu.sync_copy(adst_hbm.at[dstv], g2v)

            @pl.loop(0, _T)
            def _(i):
                a = g1v[i] + g2v[i] + aev[i]
                a = jnp.maximum(a, a * 0.2)
                exv[i] = jnp.exp(a)

            pltpu.sync_copy(exv, ex_hbm.at[pl.ds(b, _T)])
            pltpu.sync_copy(exv, den_sh.at[dstv], add=True)

        plsc.subcore_barrier()
        _write_back(den_sh, den_hbm, c, s, n)

    f = pl.kernel(
        body,
        out_type=[
            jax.ShapeDtypeStruct((e, 16), jnp.float32),
            jax.ShapeDtypeStruct((_NC, n, 16), jnp.float32),
        ],
        mesh=_mesh(),
        scratch_types=[
            pltpu.VMEM((_T,), jnp.int32),
            pltpu.VMEM((_T,), jnp.int32),
            pltpu.VMEM((_T, _L), jnp.float32),
            pltpu.VMEM((_T, _L), jnp.float32),
            pltpu.VMEM((_T, _L), jnp.float32),
            pltpu.VMEM((_T, _L), jnp.float32),
            pltpu.VMEM_SHARED((n, _L), jnp.float32),
        ],
        compiler_params=_SC_PARAMS,
    )
    return f(src, dst, asrc, adst, ae, zeros16)


def _k2_call(src, dst, htab, ex, zeros16, head):
    n = htab.shape[0]
    e = src.shape[0]
    ew = e // _NW
    nt = ew // _T

    def body(src_hbm, dst_hbm, h_hbm, ex_hbm, z_hbm, out_hbm,
             srcv, dstv, rowsv, exv, out_sh):
        c = lax.axis_index("c")
        s = lax.axis_index("s")

        @pl.when(s == 0)
        def _():
            pltpu.sync_copy(z_hbm, out_sh)
        plsc.subcore_barrier()

        base = (c * _NS + s) * ew

        @pl.loop(0, nt)
        def _(t):
            b = base + t * _T
            pltpu.sync_copy(src_hbm.at[pl.ds(b, _T)], srcv)
            pltpu.sync_copy(dst_hbm.at[pl.ds(b, _T)], dstv)
            pltpu.sync_copy(ex_hbm.at[pl.ds(b, _T)], exv)
            pltpu.sync_copy(h_hbm.at[srcv], rowsv)

            @pl.loop(0, _T)
            def _(t2):
                exw = exv[t2]
                rowsv[t2] = rowsv[t2] * exw[head]

            pltpu.sync_copy(rowsv, out_sh.at[dstv], add=True)

        plsc.subcore_barrier()
        _write_back(out_sh, out_hbm, c, s, n)

    f = pl.kernel(
        body,
        out_type=jax.ShapeDtypeStruct((_NC, n, _L), jnp.float32),
        mesh=_mesh(),
        scratch_types=[
            pltpu.VMEM((_T,), jnp.int32),
            pltpu.VMEM((_T,), jnp.int32),
            pltpu.VMEM((_T, _L), jnp.float32),
            pltpu.VMEM((_T, _L), jnp.float32),
            pltpu.VMEM_SHARED((n, _L), jnp.float32),
        ],
        compiler_params=_SC_PARAMS,
    )
    return f(src, dst, htab, ex, zeros16)


def _fold(W, a_src, a_dst):
    heads = a_src.shape[1]
    ch = a_src.shape[2]
    Wr = W.reshape(W.shape[0], heads, ch)
    Us = jnp.einsum('khc,hc->kh', Wr, a_src[0])
    Ud = jnp.einsum('khc,hc->kh', Wr, a_dst[0])
    return Us, Ud


def _pad16(u):
    k, h = u.shape
    return jnp.concatenate([u, jnp.zeros((k, 16 - h), u.dtype)], axis=1)


def kernel(x, edge_index, edge_attr,
           W1, a_src1, a_dst1, a_e1, We1, b1,
           W2, a_src2, a_dst2, a_e2, We2, b2,
           W3, a_src3, a_dst3, a_e3, We3, b3,
           Wfc, bfc):
    n = x.shape[0]
    src = edge_index[0].astype(jnp.int32)
    dst = edge_index[1].astype(jnp.int32)

    Us1, Ud1 = _fold(W1, a_src1, a_dst1)
    Us2, Ud2 = _fold(W2, a_src2, a_dst2)
    Us3, Ud3 = _fold(W3, a_src3, a_dst3)
    Ue1 = jnp.einsum('khc,hc->kh', We1.reshape(2, 4, 16), a_e1[0])
    Ue2 = jnp.einsum('khc,hc->kh', We2.reshape(2, 4, 16), a_e2[0])
    Ue3 = jnp.einsum('khc,hc->kh', We3.reshape(2, 1, 16), a_e3[0])

    wcat1 = jnp.concatenate([W1, _pad16(Us1), _pad16(Ud1)], axis=1)
    wcat2 = jnp.concatenate([W2, _pad16(Us2), _pad16(Ud2)], axis=1)
    wcat3 = jnp.concatenate([W3, _pad16(Us3), _pad16(Ud3)], axis=1)
    uecat = jnp.concatenate([_pad16(Ue1), _pad16(Ue2), _pad16(Ue3)],
                            axis=1)

    zeros16 = jnp.zeros((n, 16), jnp.float32)

    ae1, ae2, ae3 = _edge_ae(edge_attr, uecat)

    h0, h1, h2, h3_, asrc, adst = _node1(x, wcat1)
    ex1, den1 = _k1_call(src, dst, asrc, adst, ae1, zeros16)
    ps = [_k2_call(src, dst, h, ex1, zeros16, i)
          for i, h in enumerate([h0, h1, h2, h3_])]

    h0, h1, h2, h3_, asrc, adst = _node23(ps, den1, b1, wcat2, 4)
    ex2, den2 = _k1_call(src, dst, asrc, adst, ae2, zeros16)
    ps = [_k2_call(src, dst, h, ex2, zeros16, i)
          for i, h in enumerate([h0, h1, h2, h3_])]

    hh, asrc, adst = _node23(ps, den2, b2, wcat3, 1)
    ex3, den3 = _k1_call(src, dst, asrc, adst, ae3, zeros16)
    p3 = _k2_call(src, dst, hh, ex3, zeros16, 0)

    return _fc(p3, den3, b3, Wfc, bfc)


# speedup vs baseline: 57.7627x; 1.2786x over previous
import functools

import jax
import jax.numpy as jnp
from jax import lax
from jax.experimental import pallas as pl
from jax.experimental.pallas import tpu as pltpu
from jax.experimental.pallas import tpu_sc as plsc

_NC = 2
_NS = 16
_NW = _NC * _NS
_L = 16

_T = 1000


def _mesh():
    return plsc.VectorSubcoreMesh(core_axis_name="c", subcore_axis_name="s")


_SC_PARAMS = pltpu.CompilerParams(use_tc_tiling_on_sc=False)


def _split_heads(r, nh):
    outs = [r[:, 16 * h:16 * h + 16] for h in range(nh)]
    outs.append(r[:, 16 * nh:16 * nh + 16])
    outs.append(r[:, 16 * nh + 16:16 * nh + 32])
    return outs


def _node1_body(x_ref, w_ref, *out_refs):
    x = x_ref[...]
    r = x[:, 0:1] * w_ref[0:1, :] + x[:, 1:2] * w_ref[1:2, :]
    for ref, val in zip(out_refs, _split_heads(r, 4)):
        ref[...] = val


def _node1(x, wcat):
    n = x.shape[0]
    BN = 2000
    return pl.pallas_call(
        _node1_body,
        grid=(n // BN,),
        in_specs=[
            pl.BlockSpec((BN, 2), lambda i: (i, 0)),
            pl.BlockSpec((2, 96), lambda i: (0, 0)),
        ],
        out_specs=[pl.BlockSpec((BN, 16), lambda i: (i, 0))] * 6,
        out_shape=[jax.ShapeDtypeStruct((n, 16), jnp.float32)] * 6,
    )(x, wcat)


def _assemble(ps, den, b):
    d = den[0] + den[1] + 1e-16
    cols = []
    for h, p in enumerate(ps):
        cols.append((p[0] + p[1]) / d[:, h:h + 1])
    xx = jnp.concatenate(cols, axis=1)
    return jnp.maximum(xx + b, 0.0)


def _node23_body(p0, p1, p2, p3, den_ref, b_ref, w_ref, *out_refs):
    xx = _assemble([p0[...], p1[...], p2[...], p3[...]],
                   den_ref[...], b_ref[...])
    r = jnp.dot(xx, w_ref[...], preferred_element_type=jnp.float32)
    nh = (w_ref.shape[1] - 32) // 16
    for ref, val in zip(out_refs, _split_heads(r, nh)):
        ref[...] = val


def _node23(ps, den, b, wcat, nh):
    n = ps[0].shape[1]
    BN = 2000
    wcols = 16 * nh + 32
    return pl.pallas_call(
        _node23_body,
        grid=(n // BN,),
        in_specs=[pl.BlockSpec((2, BN, 16), lambda i: (0, i, 0))] * 4 + [
            pl.BlockSpec((2, BN, 16), lambda i: (0, i, 0)),
            pl.BlockSpec((1, 64), lambda i: (0, 0)),
            pl.BlockSpec((64, wcols), lambda i: (0, 0)),
        ],
        out_specs=[pl.BlockSpec((BN, 16), lambda i: (i, 0))] * (nh + 2),
        out_shape=[jax.ShapeDtypeStruct((n, 16), jnp.float32)] * (nh + 2),
    )(*ps, den, b.reshape(1, 64), wcat)


def _edge_body(ea_ref, u_ref, o1_ref, o2_ref, o3_ref):
    ea = ea_ref[...]
    r = ea[:, 0:1] * u_ref[0:1, :] + ea[:, 1:2] * u_ref[1:2, :]
    o1_ref[...] = r[:, 0:16]
    o2_ref[...] = r[:, 16:32]
    o3_ref[...] = r[:, 32:48]


def _edge_ae(ea, ucat):
    e = ea.shape[0]
    BE = 8000
    outs = [jax.ShapeDtypeStruct((e, 16), jnp.float32)] * 3
    return pl.pallas_call(
        _edge_body,
        grid=(e // BE,),
        in_specs=[
            pl.BlockSpec((BE, 2), lambda i: (i, 0)),
            pl.BlockSpec((2, 48), lambda i: (0, 0)),
        ],
        out_specs=[pl.BlockSpec((BE, 16), lambda i: (i, 0))] * 3,
        out_shape=outs,
    )(ea, ucat)


def _fc_body(p3_ref, den_ref, b3_ref, w_ref, bfc_ref, o_ref):
    acc = p3_ref[0] + p3_ref[1]
    d = den_ref[0, :, 0:1] + den_ref[1, :, 0:1] + 1e-16
    xx = jnp.maximum(acc / d + b3_ref[...], 0.0)
    o_ref[...] = jnp.dot(xx, w_ref[...],
                         preferred_element_type=jnp.float32) + bfc_ref[...]


def _fc(p3, den3, b3, wfc, bfc):
    n = p3.shape[1]
    BN = 2000
    out_dim = wfc.shape[1]
    return pl.pallas_call(
        _fc_body,
        grid=(n // BN,),
        in_specs=[
            pl.BlockSpec((2, BN, 16), lambda i: (0, i, 0)),
            pl.BlockSpec((2, BN, 16), lambda i: (0, i, 0)),
            pl.BlockSpec((1, 16), lambda i: (0, 0)),
            pl.BlockSpec((16, out_dim), lambda i: (0, 0)),
            pl.BlockSpec((1, out_dim), lambda i: (0, 0)),
        ],
        out_specs=pl.BlockSpec((BN, out_dim), lambda i: (i, 0)),
        out_shape=jax.ShapeDtypeStruct((n, out_dim), jnp.float32),
    )(p3, den3, b3.reshape(1, 16), wfc, bfc.reshape(1, out_dim))


def _write_back(sh_ref, hbm_ref, c, s, n):
    rows = (n // _NS + 7) // 8 * 8
    last = n - (_NS - 1) * rows

    @pl.when(s < _NS - 1)
    def _():
        pltpu.sync_copy(sh_ref.at[pl.ds(s * rows, rows)],
                        hbm_ref.at[c, pl.ds(s * rows, rows)])

    @pl.when(s == _NS - 1)
    def _():
        pltpu.sync_copy(sh_ref.at[pl.ds((_NS - 1) * rows, last)],
                        hbm_ref.at[c, pl.ds((_NS - 1) * rows, last)])


def _k1_call(src, dst, asrc, adst, ae, zeros16):
    n = asrc.shape[0]
    e = src.shape[0]
    ew = e // _NW
    nt = ew // _T

    def body(src_hbm, dst_hbm, asrc_hbm, adst_hbm, ae_hbm, z_hbm,
             ex_hbm, den_hbm,
             srcv, dstv, g1v, g2v, aev, exv, den_sh):
        c = lax.axis_index("c")
        s = lax.axis_index("s")

        @pl.when(s == 0)
        def _():
            pltpu.sync_copy(z_hbm, den_sh)
        plsc.subcore_barrier()

        base = (c * _NS + s) * ew

        @pl.loop(0, nt)
        def _(t):
            b = base + t * _T
            pltpu.sync_copy(src_hbm.at[pl.ds(b, _T)], srcv)
            pltpu.sync_copy(dst_hbm.at[pl.ds(b, _T)], dstv)
            pltpu.sync_copy(ae_hbm.at[pl.ds(b, _T)], aev)
            pltpu.sync_copy(asrc_hbm.at[srcv], g1v)
            pltpu.sync_copy(adst_hbm.at[dstv], g2v)

            @plsc.parallel_loop(0, _T, unroll=8)
            def _(i):
                a = g1v[i] + g2v[i] + aev[i]
                a = jnp.maximum(a, a * 0.2)
                exv[i] = jnp.exp(a)

            pltpu.sync_copy(exv, ex_hbm.at[pl.ds(b, _T)])
            pltpu.sync_copy(exv, den_sh.at[dstv], add=True)

        plsc.subcore_barrier()
        _write_back(den_sh, den_hbm, c, s, n)

    f = pl.kernel(
        body,
        out_type=[
            jax.ShapeDtypeStruct((e, 16), jnp.float32),
            jax.ShapeDtypeStruct((_NC, n, 16), jnp.float32),
        ],
        mesh=_mesh(),
        scratch_types=[
            pltpu.VMEM((_T,), jnp.int32),
            pltpu.VMEM((_T,), jnp.int32),
            pltpu.VMEM((_T, _L), jnp.float32),
            pltpu.VMEM((_T, _L), jnp.float32),
            pltpu.VMEM((_T, _L), jnp.float32),
            pltpu.VMEM((_T, _L), jnp.float32),
            pltpu.VMEM_SHARED((n, _L), jnp.float32),
        ],
        compiler_params=_SC_PARAMS,
    )
    return f(src, dst, asrc, adst, ae, zeros16)


def _k2_call(src, dst, htab, ex, zeros16, head):
    n = htab.shape[0]
    e = src.shape[0]
    ew = e // _NW
    nt = ew // _T

    def body(src_hbm, dst_hbm, h_hbm, ex_hbm, z_hbm, out_hbm,
             srcv, dstv, rowsv, exv, out_sh):
        c = lax.axis_index("c")
        s = lax.axis_index("s")

        @pl.when(s == 0)
        def _():
            pltpu.sync_copy(z_hbm, out_sh)
        plsc.subcore_barrier()

        base = (c * _NS + s) * ew

        @pl.loop(0, nt)
        def _(t):
            b = base + t * _T
            pltpu.sync_copy(src_hbm.at[pl.ds(b, _T)], srcv)
            pltpu.sync_copy(dst_hbm.at[pl.ds(b, _T)], dstv)
            pltpu.sync_copy(ex_hbm.at[pl.ds(b, _T)], exv)
            pltpu.sync_copy(h_hbm.at[srcv], rowsv)

            @plsc.parallel_loop(0, _T, unroll=8)
            def _(t2):
                exw = exv[t2]
                rowsv[t2] = rowsv[t2] * exw[head]

            pltpu.sync_copy(rowsv, out_sh.at[dstv], add=True)

        plsc.subcore_barrier()
        _write_back(out_sh, out_hbm, c, s, n)

    f = pl.kernel(
        body,
        out_type=jax.ShapeDtypeStruct((_NC, n, _L), jnp.float32),
        mesh=_mesh(),
        scratch_types=[
            pltpu.VMEM((_T,), jnp.int32),
            pltpu.VMEM((_T,), jnp.int32),
            pltpu.VMEM((_T, _L), jnp.float32),
            pltpu.VMEM((_T, _L), jnp.float32),
            pltpu.VMEM_SHARED((n, _L), jnp.float32),
        ],
        compiler_params=_SC_PARAMS,
    )
    return f(src, dst, htab, ex, zeros16)


def _fold(W, a_src, a_dst):
    heads = a_src.shape[1]
    ch = a_src.shape[2]
    Wr = W.reshape(W.shape[0], heads, ch)
    Us = jnp.einsum('khc,hc->kh', Wr, a_src[0])
    Ud = jnp.einsum('khc,hc->kh', Wr, a_dst[0])
    return Us, Ud


def _pad16(u):
    k, h = u.shape
    return jnp.concatenate([u, jnp.zeros((k, 16 - h), u.dtype)], axis=1)


def kernel(x, edge_index, edge_attr,
           W1, a_src1, a_dst1, a_e1, We1, b1,
           W2, a_src2, a_dst2, a_e2, We2, b2,
           W3, a_src3, a_dst3, a_e3, We3, b3,
           Wfc, bfc):
    n = x.shape[0]
    src = edge_index[0].astype(jnp.int32)
    dst = edge_index[1].astype(jnp.int32)

    Us1, Ud1 = _fold(W1, a_src1, a_dst1)
    Us2, Ud2 = _fold(W2, a_src2, a_dst2)
    Us3, Ud3 = _fold(W3, a_src3, a_dst3)
    Ue1 = jnp.einsum('khc,hc->kh', We1.reshape(2, 4, 16), a_e1[0])
    Ue2 = jnp.einsum('khc,hc->kh', We2.reshape(2, 4, 16), a_e2[0])
    Ue3 = jnp.einsum('khc,hc->kh', We3.reshape(2, 1, 16), a_e3[0])

    wcat1 = jnp.concatenate([W1, _pad16(Us1), _pad16(Ud1)], axis=1)
    wcat2 = jnp.concatenate([W2, _pad16(Us2), _pad16(Ud2)], axis=1)
    wcat3 = jnp.concatenate([W3, _pad16(Us3), _pad16(Ud3)], axis=1)
    uecat = jnp.concatenate([_pad16(Ue1), _pad16(Ue2), _pad16(Ue3)],
                            axis=1)

    zeros16 = jnp.zeros((n, 16), jnp.float32)

    ae1, ae2, ae3 = _edge_ae(edge_attr, uecat)

    h0, h1, h2, h3_, asrc, adst = _node1(x, wcat1)
    ex1, den1 = _k1_call(src, dst, asrc, adst, ae1, zeros16)
    ps = [_k2_call(src, dst, h, ex1, zeros16, i)
          for i, h in enumerate([h0, h1, h2, h3_])]

    h0, h1, h2, h3_, asrc, adst = _node23(ps, den1, b1, wcat2, 4)
    ex2, den2 = _k1_call(src, dst, asrc, adst, ae2, zeros16)
    ps = [_k2_call(src, dst, h, ex2, zeros16, i)
          for i, h in enumerate([h0, h1, h2, h3_])]

    hh, asrc, adst = _node23(ps, den2, b2, wcat3, 1)
    ex3, den3 = _k1_call(src, dst, asrc, adst, ae3, zeros16)
    p3 = _k2_call(src, dst, hh, ex3, zeros16, 0)

    return _fc(p3, den3, b3, Wfc, bfc)
